# Initial kernel scaffold; baseline (speedup 1.0000x reference)
import jax
import jax.numpy as jnp
from jax.experimental import pallas as pl


def kernel(embedding, nodes, neighbor_lists, unique_nodes_new_index, self_feats):
    raise NotImplementedError("write your pallas kernel here")


# speedup vs baseline: 188.1389x; 188.1389x over previous
import functools

import jax
import jax.numpy as jnp
from jax import lax
from jax.experimental import pallas as pl
from jax.experimental.pallas import tpu as pltpu
from jax.experimental.pallas import tpu_sc as plsc

NC = 2
NS = 16
L = 16


def kernel(embedding, nodes, neighbor_lists, unique_nodes_new_index, self_feats):
    del nodes, unique_nodes_new_index
    N, D = embedding.shape
    B, NB = neighbor_lists.shape
    NW = NC * NS
    BW = B // NW
    G = 4
    NG = BW // G
    GNB = G * NB
    ND = D // L

    pad = (-(N + 1)) % 8 + 1
    emb_aug = jnp.concatenate(
        [embedding, jnp.zeros((pad, D), embedding.dtype)], axis=0)
    zrow = jnp.int32(N)

    mesh = plsc.VectorSubcoreMesh(
        core_axis_name="c", subcore_axis_name="s",
        num_cores=NC, num_subcores=NS)

    @functools.partial(
        pl.kernel,
        out_type=jax.ShapeDtypeStruct((B, 2 * D), jnp.float32),
        mesh=mesh,
        compiler_params=pltpu.CompilerParams(needs_layout_passes=False),
        scratch_types=[
            pltpu.VMEM((BW, NB), jnp.int32),
            pltpu.VMEM((BW, D), jnp.float32),
            pltpu.VMEM((N,), jnp.int32),
            pltpu.VMEM((GNB,), jnp.int32),
            pltpu.VMEM((GNB, D), jnp.float32),
            pltpu.VMEM((BW, 2 * D), jnp.float32),
            pltpu.SemaphoreType.DMA,
        ],
    )
    def sc_kernel(emb_hbm, nl_hbm, self_hbm, out_hbm,
                  nl_v, self_v, table_v, idx_v, rows_v, out_v, sem):
        wid = lax.axis_index("s") * NC + lax.axis_index("c")
        base = wid * BW
        pltpu.sync_copy(nl_hbm.at[pl.ds(base, BW)], nl_v)
        pltpu.sync_copy(self_hbm.at[pl.ds(base, BW)], self_v)
        iota = lax.iota(jnp.int32, L)

        def group_body(g, carry):
            cnts = []
            for j in range(G):
                row = g * G + j
                ids0 = nl_v[row, pl.ds(0, L)]
                ids1 = nl_v[row, pl.ds(L, L)]
                tag0 = row * NB + iota
                tag1 = tag0 + L
                plsc.store_scatter(table_v, [ids0], tag0)
                plsc.store_scatter(table_v, [ids1], tag1)
                w0 = plsc.load_gather(table_v, [ids0]) == tag0
                w1 = plsc.load_gather(table_v, [ids1]) == tag1
                cnt = (plsc.all_reduce_population_count(w0)
                       + plsc.all_reduce_population_count(w1))
                cnts.append(cnt)
                idx_v[pl.ds(j * NB, L)] = jnp.where(w0, ids0, zrow)
                idx_v[pl.ds(j * NB + L, L)] = jnp.where(w1, ids1, zrow)
            pltpu.async_copy(emb_hbm.at[idx_v], rows_v, sem).wait()
            for j in range(G):
                row = g * G + j

                def acc_body(i, acc, j=j):
                    return tuple(
                        acc[d] + rows_v[j * NB + i, pl.ds(d * L, L)]
                        for d in range(ND))

                acc = lax.fori_loop(
                    0, NB, acc_body,
                    tuple(jnp.zeros((L,), jnp.float32) for _ in range(ND)))
                scale = 1.0 / cnts[j].astype(jnp.float32)
                for d in range(ND):
                    f1 = acc[d] * scale
                    out_v[row, pl.ds(d * L, L)] = f1
                    out_v[row, pl.ds(D + d * L, L)] = (
                        self_v[row, pl.ds(d * L, L)] - f1)
            return carry

        lax.fori_loop(0, NG, group_body, jnp.int32(0))
        pltpu.sync_copy(out_v, out_hbm.at[pl.ds(base, BW)])

    return sc_kernel(emb_aug, neighbor_lists, self_feats)


# speedup vs baseline: 275.6124x; 1.4649x over previous
import functools

import jax
import jax.numpy as jnp
from jax import lax
from jax.experimental import pallas as pl
from jax.experimental.pallas import tpu as pltpu
from jax.experimental.pallas import tpu_sc as plsc

NC = 2
NS = 16
L = 16


def kernel(embedding, nodes, neighbor_lists, unique_nodes_new_index, self_feats):
    del nodes, unique_nodes_new_index
    N, D = embedding.shape
    B, NB = neighbor_lists.shape
    NW = NC * NS
    BW = B // NW
    G = 4
    NG = BW // G
    GNB = G * NB
    ND = D // L

    pad = (-(N + 1)) % 8 + 1
    emb_aug = jnp.concatenate(
        [embedding, jnp.zeros((pad, D), embedding.dtype)], axis=0)
    zrow = jnp.int32(N)

    mesh = plsc.VectorSubcoreMesh(
        core_axis_name="c", subcore_axis_name="s",
        num_cores=NC, num_subcores=NS)

    @functools.partial(
        pl.kernel,
        out_type=jax.ShapeDtypeStruct((B, 2 * D), jnp.float32),
        mesh=mesh,
        compiler_params=pltpu.CompilerParams(needs_layout_passes=False),
        scratch_types=[
            pltpu.VMEM((BW, NB), jnp.int32),
            pltpu.VMEM((BW, D), jnp.float32),
            pltpu.VMEM((N,), jnp.int32),
            pltpu.VMEM((GNB,), jnp.int32),
            pltpu.VMEM((GNB,), jnp.int32),
            pltpu.VMEM((GNB, D), jnp.float32),
            pltpu.VMEM((GNB, D), jnp.float32),
            pltpu.VMEM((BW, 2 * D), jnp.float32),
            pltpu.SemaphoreType.DMA,
            pltpu.SemaphoreType.DMA,
        ],
    )
    def sc_kernel(emb_hbm, nl_hbm, self_hbm, out_hbm,
                  nl_v, self_v, table_v, idx0_v, idx1_v, rows0_v, rows1_v,
                  out_v, sem0, sem1):
        wid = lax.axis_index("s") * NC + lax.axis_index("c")
        base = wid * BW
        pltpu.sync_copy(nl_hbm.at[pl.ds(base, BW)], nl_v)
        pltpu.sync_copy(self_hbm.at[pl.ds(base, BW)], self_v)
        iota = lax.iota(jnp.int32, L)

        def prep(g, idx_v):
            for j in range(G):
                row = g * G + j
                ids0 = nl_v[row, pl.ds(0, L)]
                ids1 = nl_v[row, pl.ds(L, L)]
                tag0 = row * NB + iota
                tag1 = tag0 + L
                plsc.store_scatter(table_v, [ids0], tag0)
                plsc.store_scatter(table_v, [ids1], tag1)
                w0 = plsc.load_gather(table_v, [ids0]) == tag0
                w1 = plsc.load_gather(table_v, [ids1]) == tag1
                cnt = (plsc.all_reduce_population_count(w0)
                       + plsc.all_reduce_population_count(w1))
                idx_v[pl.ds(j * NB, L)] = jnp.where(w0, ids0, zrow)
                idx_v[pl.ds(j * NB + L, L)] = jnp.where(w1, ids1, zrow)
                out_v[row, pl.ds(0, L)] = jnp.broadcast_to(
                    cnt.astype(jnp.float32), (L,))

        def fire(idx_v, rows_v, sem):
            pltpu.async_copy(emb_hbm.at[idx_v], rows_v, sem)

        def drain(idx_v, rows_v, sem):
            pltpu.make_async_copy(emb_hbm.at[idx_v], rows_v, sem).wait()

        def accum(g, rows_v):
            for j in range(G):
                row = g * G + j
                scale = 1.0 / out_v[row, pl.ds(0, L)]

                def acc_body(i, acc, j=j):
                    return tuple(
                        acc[d] + rows_v[j * NB + i, pl.ds(d * L, L)]
                        for d in range(ND))

                acc = lax.fori_loop(
                    0, NB, acc_body,
                    tuple(jnp.zeros((L,), jnp.float32) for _ in range(ND)))
                for d in range(ND):
                    f1 = acc[d] * scale
                    out_v[row, pl.ds(d * L, L)] = f1
                    out_v[row, pl.ds(D + d * L, L)] = (
                        self_v[row, pl.ds(d * L, L)] - f1)

        prep(0, idx0_v)
        fire(idx0_v, rows0_v, sem0)
        prep(1, idx1_v)
        fire(idx1_v, rows1_v, sem1)

        def pipe_body(k, carry):
            g0 = 2 * k
            drain(idx0_v, rows0_v, sem0)
            accum(g0, rows0_v)
            prep(g0 + 2, idx0_v)
            fire(idx0_v, rows0_v, sem0)
            drain(idx1_v, rows1_v, sem1)
            accum(g0 + 1, rows1_v)
            prep(g0 + 3, idx1_v)
            fire(idx1_v, rows1_v, sem1)
            return carry

        lax.fori_loop(0, NG // 2 - 1, pipe_body, jnp.int32(0))
        drain(idx0_v, rows0_v, sem0)
        accum(NG - 2, rows0_v)
        drain(idx1_v, rows1_v, sem1)
        accum(NG - 1, rows1_v)
        pltpu.sync_copy(out_v, out_hbm.at[pl.ds(base, BW)])

    return sc_kernel(emb_aug, neighbor_lists, self_feats)
